# baseline (speedup 1.0000x reference)
import functools

import jax
import jax.numpy as jnp
from jax import lax
from jax.experimental import pallas as pl
from jax.experimental.pallas import tpu as pltpu
from jax.experimental.pallas import tpu_sc as plsc

D = 128
NC = 2
NS = 16
CHUNK = 128
NSLOT = 4
SPLIT0 = 0.2


def _sc_mesh():
    return plsc.VectorSubcoreMesh(
        core_axis_name="c", subcore_axis_name="s", num_cores=NC, num_subcores=NS
    )


def _deg_body(n_pad, n_chunks, edges_hbm, deg_hbm, deg_sh, idx_v, ones_v, zbuf):
    c = lax.axis_index("c")
    s = lax.axis_index("s")
    rows = n_pad // NS
    base = s * rows

    ones16 = jnp.full((16,), 1.0, jnp.float32)
    zero16 = jnp.zeros((16,), jnp.float32)

    def fill_ones(i, _):
        ones_v[i, :] = ones16
        return 0

    lax.fori_loop(0, CHUNK, fill_ones, 0)

    def fill_zero(i, _):
        zbuf[i, :] = zero16
        return 0

    lax.fori_loop(0, rows, fill_zero, 0)

    pltpu.sync_copy(zbuf, deg_sh.at[pl.ds(base, rows)])
    plsc.subcore_barrier()

    pltpu.sync_copy(edges_hbm.at[c, s, pl.ds(0, idx_v.shape[0]), pl.ds(1, 1)],
                    idx_v)

    def chunk(g, _):
        pltpu.sync_copy(ones_v, deg_sh.at[idx_v.at[g, 0]], add=True)
        return 0

    lax.fori_loop(0, n_chunks, chunk, 0)
    plsc.subcore_barrier()

    pltpu.sync_copy(deg_sh.at[pl.ds(base, rows)], deg_hbm.at[c, pl.ds(base, rows)])


def _sc_deg(edges5, n_pad):
    n_chunks = edges5.shape[2]
    rows = n_pad // NS
    body = functools.partial(_deg_body, n_pad, n_chunks)
    return pl.kernel(
        body,
        out_type=jax.ShapeDtypeStruct((NC, n_pad, 16), jnp.float32),
        mesh=_sc_mesh(),
        scratch_types=[
            pltpu.VMEM_SHARED((n_pad, 16), jnp.float32),
            pltpu.VMEM((n_chunks, 1, CHUNK), jnp.int32),
            pltpu.VMEM((CHUNK, 16), jnp.float32),
            pltpu.VMEM((rows, 16), jnp.float32),
        ],
    )(edges5)


def _h_body(x_ref, w_ref, d0_ref, d1_ref, h_ref):
    deg = d0_ref[:, 0:1] + d1_ref[:, 0:1] + 1.0
    dinv = lax.rsqrt(deg)
    h = jnp.dot(
        x_ref[...],
        w_ref[...],
        preferred_element_type=jnp.float32,
        precision=lax.Precision.HIGHEST,
    )
    h_ref[...] = h * dinv


def _tc_h(x_pad, w, deg0, deg1, n_pad, bn=1024):
    return pl.pallas_call(
        _h_body,
        grid=(n_pad // bn,),
        in_specs=[
            pl.BlockSpec((bn, D), lambda i: (i, 0)),
            pl.BlockSpec((D, D), lambda i: (0, 0)),
            pl.BlockSpec((bn, 16), lambda i: (i, 0)),
            pl.BlockSpec((bn, 16), lambda i: (i, 0)),
        ],
        out_specs=pl.BlockSpec((bn, D), lambda i: (i, 0)),
        out_shape=jax.ShapeDtypeStruct((n_pad, D), jnp.float32),
    )(x_pad, w, deg0, deg1)


def _main_body(n_pad, nch0, nch1, edges_hbm, h_hbm, part_hbm,
               out_sh, eg, bufs, gsems, isems):
    c = lax.axis_index("c")
    s = lax.axis_index("s")
    n_chunks = nch0 + (nch1 - nch0) * c
    rows = n_pad // NS
    base = s * rows
    edges = edges_hbm.at[c, s]

    zero16 = jnp.zeros((16,), jnp.float32)

    def fill_zero(i, _):
        def inner(j, _):
            bufs[0][i, pl.ds(j * 16, 16)] = zero16
            return 0

        lax.fori_loop(0, D // 16, inner, 0)
        return 0

    lax.fori_loop(0, CHUNK, fill_zero, 0)

    for r in range(rows // CHUNK):
        pltpu.sync_copy(bufs[0], out_sh.at[pl.ds(base + r * CHUNK, CHUNK)])
    plsc.subcore_barrier()

    def start_idx(k, q):
        pltpu.async_copy(edges.at[pl.ds(k, 1)], eg.at[pl.ds(q, 1)], isems[q])

    def wait_idx(q):
        pltpu.make_async_copy(edges.at[pl.ds(0, 1)], eg.at[pl.ds(q, 1)],
                              isems[q]).wait()

    def launch(q, p):
        pltpu.async_copy(h_hbm.at[eg.at[q, 0]], bufs[p], gsems[p])

    def drain(q, p):
        pltpu.make_async_copy(h_hbm.at[eg.at[q, 0]], bufs[p], gsems[p]).wait()
        pltpu.sync_copy(bufs[p], out_sh.at[eg.at[q, 1]], add=True)

    for q in range(NSLOT):
        start_idx(q, q)
    for q in range(2):
        wait_idx(q)
        launch(q, q)

    def quad(j, _):
        k4 = j * NSLOT
        for t in range(NSLOT):
            p = t % 2
            kt = k4 + t
            drain(t, p)

            @pl.when(kt + NSLOT < n_chunks)
            def _():
                start_idx(kt + NSLOT, t)

            @pl.when(kt + 2 < n_chunks)
            def _():
                wait_idx((t + 2) % NSLOT)
                launch((t + 2) % NSLOT, p)

        return 0

    lax.fori_loop(0, n_chunks // NSLOT, quad, 0)
    plsc.subcore_barrier()

    for r in range(rows // CHUNK):
        p = r % 2
        pltpu.sync_copy(out_sh.at[pl.ds(base + r * CHUNK, CHUNK)], bufs[p])
        pltpu.sync_copy(bufs[p], part_hbm.at[c, pl.ds(base + r * CHUNK, CHUNK)])


def _sc_main(edges5, hprime, n_pad, nch0, nch1):
    n_chunks = edges5.shape[2]
    body = functools.partial(_main_body, n_pad, nch0, nch1)
    return pl.kernel(
        body,
        out_type=jax.ShapeDtypeStruct((NC, n_pad, D), jnp.float32),
        mesh=_sc_mesh(),
        scratch_types=[
            pltpu.VMEM_SHARED((n_pad, D), jnp.float32),
            pltpu.VMEM((NSLOT, 2, CHUNK), jnp.int32),
            [pltpu.VMEM((CHUNK, D), jnp.float32) for _ in range(2)],
            [pltpu.SemaphoreType.DMA for _ in range(2)],
            [pltpu.SemaphoreType.DMA for _ in range(NSLOT)],
        ],
    )(edges5, hprime)


def _final_body(p_ref, h_ref, d0_ref, d1_ref, b_ref, pw_ref, o_ref):
    deg = d0_ref[:, 0:1] + d1_ref[:, 0:1] + 1.0
    dinv = lax.rsqrt(deg)
    acc = (p_ref[0] + p_ref[1] + h_ref[...]) * dinv + b_ref[...]
    pw = pw_ref[0, 0]
    o_ref[...] = jnp.where(acc >= 0, acc, pw * acc)


def _tc_final(parts, hprime, deg0, deg1, b2, pw2, n_pad, bn=1024):
    return pl.pallas_call(
        _final_body,
        grid=(n_pad // bn,),
        in_specs=[
            pl.BlockSpec((NC, bn, D), lambda i: (0, i, 0)),
            pl.BlockSpec((bn, D), lambda i: (i, 0)),
            pl.BlockSpec((bn, 16), lambda i: (i, 0)),
            pl.BlockSpec((bn, 16), lambda i: (i, 0)),
            pl.BlockSpec((1, D), lambda i: (0, 0)),
            pl.BlockSpec((1, 1), lambda i: (0, 0)),
        ],
        out_specs=pl.BlockSpec((bn, D), lambda i: (i, 0)),
        out_shape=jax.ShapeDtypeStruct((n_pad, D), jnp.float32),
    )(parts, hprime, deg0, deg1, b2, pw2)


def kernel(x, edge_index, W, b, prelu_weight):
    n = x.shape[0]
    e = edge_index.shape[1]
    tile_quant = NS * CHUNK
    n_pad = ((n + 1 + tile_quant - 1) // tile_quant) * tile_quant
    tile_chunk = NS * CHUNK
    total_chunks = -(-e // tile_chunk)
    total_chunks = -(-total_chunks // (2 * NSLOT)) * (2 * NSLOT)
    nch0 = max(NSLOT, int(round(SPLIT0 * total_chunks / NSLOT)) * NSLOT)
    nch1 = total_chunks - nch0
    nch = max(nch0, nch1)
    e_pad = total_chunks * tile_chunk

    src = edge_index[0].astype(jnp.int32)
    dst = edge_index[1].astype(jnp.int32)
    pad = jnp.full((e_pad - e,), n, jnp.int32)

    def per_core(flat):
        c0 = flat[: nch0 * tile_chunk].reshape(NS, nch0, 1, CHUNK)
        c1 = flat[nch0 * tile_chunk:].reshape(NS, nch1, 1, CHUNK)
        c0 = jnp.pad(c0, ((0, 0), (0, nch - nch0), (0, 0), (0, 0)),
                     constant_values=n)
        c1 = jnp.pad(c1, ((0, 0), (0, nch - nch1), (0, 0), (0, 0)),
                     constant_values=n)
        return jnp.stack([c0, c1])

    src5 = per_core(jnp.concatenate([src, pad]))
    dst5 = per_core(jnp.concatenate([dst, pad]))
    edges5 = jnp.concatenate([src5, dst5], axis=3)
    x_pad = jnp.pad(x, ((0, n_pad - n), (0, 0)))

    deg = _sc_deg(edges5, n_pad)
    hprime = _tc_h(x_pad, W, deg[0], deg[1], n_pad)
    parts = _sc_main(edges5, hprime, n_pad, nch0, nch1)
    out_pad = _tc_final(
        parts, hprime, deg[0], deg[1],
        b.reshape(1, D), prelu_weight.reshape(1, 1).astype(jnp.float32), n_pad,
    )
    return out_pad[:n]


# speedup vs baseline: 1.0465x; 1.0465x over previous
import functools

import jax
import jax.numpy as jnp
from jax import lax
from jax.experimental import pallas as pl
from jax.experimental.pallas import tpu as pltpu
from jax.experimental.pallas import tpu_sc as plsc

D = 128
NC = 2
NS = 16
CHUNK = 128
NSLOT = 4
SPLIT0 = 0.8


def _sc_mesh():
    return plsc.VectorSubcoreMesh(
        core_axis_name="c", subcore_axis_name="s", num_cores=NC, num_subcores=NS
    )


def _deg_body(n_pad, n_chunks, edges_hbm, deg_hbm, deg_sh, idx_v, ones_v, zbuf):
    c = lax.axis_index("c")
    s = lax.axis_index("s")
    rows = n_pad // NS
    base = s * rows

    ones16 = jnp.full((16,), 1.0, jnp.float32)
    zero16 = jnp.zeros((16,), jnp.float32)

    def fill_ones(i, _):
        ones_v[i, :] = ones16
        return 0

    lax.fori_loop(0, CHUNK, fill_ones, 0)

    def fill_zero(i, _):
        zbuf[i, :] = zero16
        return 0

    lax.fori_loop(0, rows, fill_zero, 0)

    pltpu.sync_copy(zbuf, deg_sh.at[pl.ds(base, rows)])
    plsc.subcore_barrier()

    pltpu.sync_copy(edges_hbm.at[c, s, pl.ds(0, idx_v.shape[0]), pl.ds(1, 1)],
                    idx_v)

    def chunk(g, _):
        pltpu.sync_copy(ones_v, deg_sh.at[idx_v.at[g, 0]], add=True)
        return 0

    lax.fori_loop(0, n_chunks, chunk, 0)
    plsc.subcore_barrier()

    pltpu.sync_copy(deg_sh.at[pl.ds(base, rows)], deg_hbm.at[c, pl.ds(base, rows)])


def _sc_deg(edges5, n_pad):
    n_chunks = edges5.shape[2]
    rows = n_pad // NS
    body = functools.partial(_deg_body, n_pad, n_chunks)
    return pl.kernel(
        body,
        out_type=jax.ShapeDtypeStruct((NC, n_pad, 16), jnp.float32),
        mesh=_sc_mesh(),
        scratch_types=[
            pltpu.VMEM_SHARED((n_pad, 16), jnp.float32),
            pltpu.VMEM((n_chunks, 1, CHUNK), jnp.int32),
            pltpu.VMEM((CHUNK, 16), jnp.float32),
            pltpu.VMEM((rows, 16), jnp.float32),
        ],
    )(edges5)


def _h_body(x_ref, w_ref, d0_ref, d1_ref, h_ref):
    deg = d0_ref[:, 0:1] + d1_ref[:, 0:1] + 1.0
    dinv = lax.rsqrt(deg)
    h = jnp.dot(
        x_ref[...],
        w_ref[...],
        preferred_element_type=jnp.float32,
        precision=lax.Precision.HIGHEST,
    )
    h_ref[...] = h * dinv


def _tc_h(x_pad, w, deg0, deg1, n_pad, bn=1024):
    return pl.pallas_call(
        _h_body,
        grid=(n_pad // bn,),
        in_specs=[
            pl.BlockSpec((bn, D), lambda i: (i, 0)),
            pl.BlockSpec((D, D), lambda i: (0, 0)),
            pl.BlockSpec((bn, 16), lambda i: (i, 0)),
            pl.BlockSpec((bn, 16), lambda i: (i, 0)),
        ],
        out_specs=pl.BlockSpec((bn, D), lambda i: (i, 0)),
        out_shape=jax.ShapeDtypeStruct((n_pad, D), jnp.float32),
    )(x_pad, w, deg0, deg1)


def _main_body(n_pad, nch0, nch1, edges_hbm, h_hbm, part_hbm,
               out_sh, eg, bufs, gsems, isems):
    c = lax.axis_index("c")
    s = lax.axis_index("s")
    n_chunks = nch0 + (nch1 - nch0) * c
    rows = n_pad // NS
    base = s * rows
    edges = edges_hbm.at[c, s]

    zero16 = jnp.zeros((16,), jnp.float32)

    def fill_zero(i, _):
        def inner(j, _):
            bufs[0][i, pl.ds(j * 16, 16)] = zero16
            return 0

        lax.fori_loop(0, D // 16, inner, 0)
        return 0

    lax.fori_loop(0, CHUNK, fill_zero, 0)

    for r in range(rows // CHUNK):
        pltpu.sync_copy(bufs[0], out_sh.at[pl.ds(base + r * CHUNK, CHUNK)])
    plsc.subcore_barrier()

    def start_idx(k, q):
        pltpu.async_copy(edges.at[pl.ds(k, 1)], eg.at[pl.ds(q, 1)], isems[q])

    def wait_idx(q):
        pltpu.make_async_copy(edges.at[pl.ds(0, 1)], eg.at[pl.ds(q, 1)],
                              isems[q]).wait()

    def launch(q, p):
        pltpu.async_copy(h_hbm.at[eg.at[q, 0]], bufs[p], gsems[p])

    def drain(q, p):
        pltpu.make_async_copy(h_hbm.at[eg.at[q, 0]], bufs[p], gsems[p]).wait()
        pltpu.sync_copy(bufs[p], out_sh.at[eg.at[q, 1]], add=True)

    for q in range(NSLOT):
        start_idx(q, q)
    for q in range(2):
        wait_idx(q)
        launch(q, q)

    def quad(j, _):
        k4 = j * NSLOT
        for t in range(NSLOT):
            p = t % 2
            kt = k4 + t
            drain(t, p)

            @pl.when(kt + NSLOT < n_chunks)
            def _():
                start_idx(kt + NSLOT, t)

            @pl.when(kt + 2 < n_chunks)
            def _():
                wait_idx((t + 2) % NSLOT)
                launch((t + 2) % NSLOT, p)

        return 0

    lax.fori_loop(0, n_chunks // NSLOT, quad, 0)
    plsc.subcore_barrier()

    for r in range(rows // CHUNK):
        p = r % 2
        pltpu.sync_copy(out_sh.at[pl.ds(base + r * CHUNK, CHUNK)], bufs[p])
        pltpu.sync_copy(bufs[p], part_hbm.at[c, pl.ds(base + r * CHUNK, CHUNK)])


def _sc_main(edges5, hprime, n_pad, nch0, nch1):
    n_chunks = edges5.shape[2]
    body = functools.partial(_main_body, n_pad, nch0, nch1)
    return pl.kernel(
        body,
        out_type=jax.ShapeDtypeStruct((NC, n_pad, D), jnp.float32),
        mesh=_sc_mesh(),
        scratch_types=[
            pltpu.VMEM_SHARED((n_pad, D), jnp.float32),
            pltpu.VMEM((NSLOT, 2, CHUNK), jnp.int32),
            [pltpu.VMEM((CHUNK, D), jnp.float32) for _ in range(2)],
            [pltpu.SemaphoreType.DMA for _ in range(2)],
            [pltpu.SemaphoreType.DMA for _ in range(NSLOT)],
        ],
    )(edges5, hprime)


def _final_body(p_ref, h_ref, d0_ref, d1_ref, b_ref, pw_ref, o_ref):
    deg = d0_ref[:, 0:1] + d1_ref[:, 0:1] + 1.0
    dinv = lax.rsqrt(deg)
    acc = (p_ref[0] + p_ref[1] + h_ref[...]) * dinv + b_ref[...]
    pw = pw_ref[0, 0]
    o_ref[...] = jnp.where(acc >= 0, acc, pw * acc)


def _tc_final(parts, hprime, deg0, deg1, b2, pw2, n_pad, bn=1024):
    return pl.pallas_call(
        _final_body,
        grid=(n_pad // bn,),
        in_specs=[
            pl.BlockSpec((NC, bn, D), lambda i: (0, i, 0)),
            pl.BlockSpec((bn, D), lambda i: (i, 0)),
            pl.BlockSpec((bn, 16), lambda i: (i, 0)),
            pl.BlockSpec((bn, 16), lambda i: (i, 0)),
            pl.BlockSpec((1, D), lambda i: (0, 0)),
            pl.BlockSpec((1, 1), lambda i: (0, 0)),
        ],
        out_specs=pl.BlockSpec((bn, D), lambda i: (i, 0)),
        out_shape=jax.ShapeDtypeStruct((n_pad, D), jnp.float32),
    )(parts, hprime, deg0, deg1, b2, pw2)


def kernel(x, edge_index, W, b, prelu_weight):
    n = x.shape[0]
    e = edge_index.shape[1]
    tile_quant = NS * CHUNK
    n_pad = ((n + 1 + tile_quant - 1) // tile_quant) * tile_quant
    tile_chunk = NS * CHUNK
    total_chunks = -(-e // tile_chunk)
    total_chunks = -(-total_chunks // (2 * NSLOT)) * (2 * NSLOT)
    nch0 = max(NSLOT, int(round(SPLIT0 * total_chunks / NSLOT)) * NSLOT)
    nch1 = total_chunks - nch0
    nch = max(nch0, nch1)
    e_pad = total_chunks * tile_chunk

    src = edge_index[0].astype(jnp.int32)
    dst = edge_index[1].astype(jnp.int32)
    pad = jnp.full((e_pad - e,), n, jnp.int32)

    def per_core(flat):
        c0 = flat[: nch0 * tile_chunk].reshape(NS, nch0, 1, CHUNK)
        c1 = flat[nch0 * tile_chunk:].reshape(NS, nch1, 1, CHUNK)
        c0 = jnp.pad(c0, ((0, 0), (0, nch - nch0), (0, 0), (0, 0)),
                     constant_values=n)
        c1 = jnp.pad(c1, ((0, 0), (0, nch - nch1), (0, 0), (0, 0)),
                     constant_values=n)
        return jnp.stack([c0, c1])

    src5 = per_core(jnp.concatenate([src, pad]))
    dst5 = per_core(jnp.concatenate([dst, pad]))
    edges5 = jnp.concatenate([src5, dst5], axis=3)
    x_pad = jnp.pad(x, ((0, n_pad - n), (0, 0)))

    deg = _sc_deg(edges5, n_pad)
    hprime = _tc_h(x_pad, W, deg[0], deg[1], n_pad)
    parts = _sc_main(edges5, hprime, n_pad, nch0, nch1)
    out_pad = _tc_final(
        parts, hprime, deg[0], deg[1],
        b.reshape(1, D), prelu_weight.reshape(1, 1).astype(jnp.float32), n_pad,
    )
    return out_pad[:n]


# speedup vs baseline: 1.4612x; 1.3963x over previous
import functools

import jax
import jax.numpy as jnp
from jax import lax
from jax.experimental import pallas as pl
from jax.experimental.pallas import tpu as pltpu
from jax.experimental.pallas import tpu_sc as plsc

D = 128
NC = 2
NS = 16
CHUNK = 128
NSLOT = 4
SPLIT0 = 0.5


def _sc_mesh():
    return plsc.VectorSubcoreMesh(
        core_axis_name="c", subcore_axis_name="s", num_cores=NC, num_subcores=NS
    )


def _deg_body(n_pad, n_chunks, edges_hbm, deg_hbm, deg_sh, idx_v, ones_v, zbuf):
    c = lax.axis_index("c")
    s = lax.axis_index("s")
    rows = n_pad // NS
    base = s * rows

    ones16 = jnp.full((16,), 1.0, jnp.float32)
    zero16 = jnp.zeros((16,), jnp.float32)

    def fill_ones(i, _):
        ones_v[i, :] = ones16
        return 0

    lax.fori_loop(0, CHUNK, fill_ones, 0)

    def fill_zero(i, _):
        zbuf[i, :] = zero16
        return 0

    lax.fori_loop(0, rows, fill_zero, 0)

    pltpu.sync_copy(zbuf, deg_sh.at[pl.ds(base, rows)])
    plsc.subcore_barrier()

    pltpu.sync_copy(edges_hbm.at[c, s, pl.ds(0, idx_v.shape[0]), pl.ds(1, 1)],
                    idx_v)

    def chunk(g, _):
        pltpu.sync_copy(ones_v, deg_sh.at[idx_v.at[g, 0]], add=True)
        return 0

    lax.fori_loop(0, n_chunks, chunk, 0)
    plsc.subcore_barrier()

    pltpu.sync_copy(deg_sh.at[pl.ds(base, rows)], deg_hbm.at[c, pl.ds(base, rows)])


def _sc_deg(edges5, n_pad):
    n_chunks = edges5.shape[2]
    rows = n_pad // NS
    body = functools.partial(_deg_body, n_pad, n_chunks)
    return pl.kernel(
        body,
        out_type=jax.ShapeDtypeStruct((NC, n_pad, 16), jnp.float32),
        mesh=_sc_mesh(),
        scratch_types=[
            pltpu.VMEM_SHARED((n_pad, 16), jnp.float32),
            pltpu.VMEM((n_chunks, 1, CHUNK), jnp.int32),
            pltpu.VMEM((CHUNK, 16), jnp.float32),
            pltpu.VMEM((rows, 16), jnp.float32),
        ],
    )(edges5)


def _h_body(x_ref, w_ref, d0_ref, d1_ref, h_ref):
    deg = d0_ref[:, 0:1] + d1_ref[:, 0:1] + 1.0
    dinv = lax.rsqrt(deg)
    h = jnp.dot(
        x_ref[...],
        w_ref[...],
        preferred_element_type=jnp.float32,
        precision=lax.Precision.HIGHEST,
    )
    h_ref[...] = h * dinv


def _tc_h(x_pad, w, deg0, deg1, n_pad, bn=1024):
    return pl.pallas_call(
        _h_body,
        grid=(n_pad // bn,),
        in_specs=[
            pl.BlockSpec((bn, D), lambda i: (i, 0)),
            pl.BlockSpec((D, D), lambda i: (0, 0)),
            pl.BlockSpec((bn, 16), lambda i: (i, 0)),
            pl.BlockSpec((bn, 16), lambda i: (i, 0)),
        ],
        out_specs=pl.BlockSpec((bn, D), lambda i: (i, 0)),
        out_shape=jax.ShapeDtypeStruct((n_pad, D), jnp.float32),
    )(x_pad, w, deg0, deg1)


def _main_body(n_pad, nch0, nch1, edges_hbm, h_hbm, part_hbm,
               out_sh, eg, bufs, gsems, isems, ssems):
    c = lax.axis_index("c")
    s = lax.axis_index("s")
    n_chunks = nch0 + (nch1 - nch0) * c
    rows = n_pad // NS
    base = s * rows
    edges = edges_hbm.at[c, s]

    zero16 = jnp.zeros((16,), jnp.float32)

    def fill_zero(i, _):
        def inner(j, _):
            bufs[0][i, pl.ds(j * 16, 16)] = zero16
            return 0

        lax.fori_loop(0, D // 16, inner, 0)
        return 0

    lax.fori_loop(0, CHUNK, fill_zero, 0)

    for r in range(rows // CHUNK):
        pltpu.sync_copy(bufs[0], out_sh.at[pl.ds(base + r * CHUNK, CHUNK)])
    plsc.subcore_barrier()

    def start_idx(k, q):
        pltpu.async_copy(edges.at[pl.ds(k, 1)], eg.at[pl.ds(q, 1)], isems[q])

    def wait_idx(q):
        pltpu.make_async_copy(edges.at[pl.ds(0, 1)], eg.at[pl.ds(q, 1)],
                              isems[q]).wait()

    def launch(q, p):
        pltpu.async_copy(h_hbm.at[eg.at[q, 0]], bufs[p], gsems[p])

    def wait_gather(q, p):
        pltpu.make_async_copy(h_hbm.at[eg.at[q, 0]], bufs[p], gsems[p]).wait()

    def scatter(q, p):
        pltpu.async_copy(bufs[p], out_sh.at[eg.at[q, 1]], ssems[p], add=True)

    def wait_scatter(q, p):
        pltpu.make_async_copy(bufs[p], out_sh.at[eg.at[q, 1]], ssems[p]).wait()

    for q in range(NSLOT):
        start_idx(q, q)
    for q in range(2):
        wait_idx(q)
        launch(q, q)

    def pair(kt, q0):
        wait_gather(q0, 0)
        scatter(q0, 0)
        wait_gather(q0 + 1, 1)
        scatter(q0 + 1, 1)

        wait_scatter(q0, 0)

        @pl.when(kt + 2 < n_chunks)
        def _():
            wait_idx((q0 + 2) % NSLOT)
            launch((q0 + 2) % NSLOT, 0)

        @pl.when(kt + NSLOT < n_chunks)
        def _():
            start_idx(kt + NSLOT, q0)

        wait_scatter(q0 + 1, 1)

        @pl.when(kt + 3 < n_chunks)
        def _():
            wait_idx((q0 + 3) % NSLOT)
            launch((q0 + 3) % NSLOT, 1)

        @pl.when(kt + NSLOT + 1 < n_chunks)
        def _():
            start_idx(kt + NSLOT + 1, q0 + 1)

    def quad(j, _):
        k4 = j * NSLOT
        pair(k4, 0)
        pair(k4 + 2, 2)
        return 0

    lax.fori_loop(0, n_chunks // NSLOT, quad, 0)
    plsc.subcore_barrier()

    for r in range(rows // CHUNK):
        p = r % 2
        pltpu.sync_copy(out_sh.at[pl.ds(base + r * CHUNK, CHUNK)], bufs[p])
        pltpu.sync_copy(bufs[p], part_hbm.at[c, pl.ds(base + r * CHUNK, CHUNK)])


def _sc_main(edges5, hprime, n_pad, nch0, nch1):
    n_chunks = edges5.shape[2]
    body = functools.partial(_main_body, n_pad, nch0, nch1)
    return pl.kernel(
        body,
        out_type=jax.ShapeDtypeStruct((NC, n_pad, D), jnp.float32),
        mesh=_sc_mesh(),
        scratch_types=[
            pltpu.VMEM_SHARED((n_pad, D), jnp.float32),
            pltpu.VMEM((NSLOT, 2, CHUNK), jnp.int32),
            [pltpu.VMEM((CHUNK, D), jnp.float32) for _ in range(2)],
            [pltpu.SemaphoreType.DMA for _ in range(2)],
            [pltpu.SemaphoreType.DMA for _ in range(NSLOT)],
            [pltpu.SemaphoreType.DMA for _ in range(2)],
        ],
    )(edges5, hprime)


def _final_body(p_ref, h_ref, d0_ref, d1_ref, b_ref, pw_ref, o_ref):
    deg = d0_ref[:, 0:1] + d1_ref[:, 0:1] + 1.0
    dinv = lax.rsqrt(deg)
    acc = (p_ref[0] + p_ref[1] + h_ref[...]) * dinv + b_ref[...]
    pw = pw_ref[0, 0]
    o_ref[...] = jnp.where(acc >= 0, acc, pw * acc)


def _tc_final(parts, hprime, deg0, deg1, b2, pw2, n_pad, bn=1024):
    return pl.pallas_call(
        _final_body,
        grid=(n_pad // bn,),
        in_specs=[
            pl.BlockSpec((NC, bn, D), lambda i: (0, i, 0)),
            pl.BlockSpec((bn, D), lambda i: (i, 0)),
            pl.BlockSpec((bn, 16), lambda i: (i, 0)),
            pl.BlockSpec((bn, 16), lambda i: (i, 0)),
            pl.BlockSpec((1, D), lambda i: (0, 0)),
            pl.BlockSpec((1, 1), lambda i: (0, 0)),
        ],
        out_specs=pl.BlockSpec((bn, D), lambda i: (i, 0)),
        out_shape=jax.ShapeDtypeStruct((n_pad, D), jnp.float32),
    )(parts, hprime, deg0, deg1, b2, pw2)


def kernel(x, edge_index, W, b, prelu_weight):
    n = x.shape[0]
    e = edge_index.shape[1]
    tile_quant = NS * CHUNK
    n_pad = ((n + 1 + tile_quant - 1) // tile_quant) * tile_quant
    tile_chunk = NS * CHUNK
    total_chunks = -(-e // tile_chunk)
    total_chunks = -(-total_chunks // (2 * NSLOT)) * (2 * NSLOT)
    nch0 = max(NSLOT, int(round(SPLIT0 * total_chunks / NSLOT)) * NSLOT)
    nch1 = total_chunks - nch0
    nch = max(nch0, nch1)
    e_pad = total_chunks * tile_chunk

    src = edge_index[0].astype(jnp.int32)
    dst = edge_index[1].astype(jnp.int32)
    pad = jnp.full((e_pad - e,), n, jnp.int32)

    def per_core(flat):
        c0 = flat[: nch0 * tile_chunk].reshape(NS, nch0, 1, CHUNK)
        c1 = flat[nch0 * tile_chunk:].reshape(NS, nch1, 1, CHUNK)
        c0 = jnp.pad(c0, ((0, 0), (0, nch - nch0), (0, 0), (0, 0)),
                     constant_values=n)
        c1 = jnp.pad(c1, ((0, 0), (0, nch - nch1), (0, 0), (0, 0)),
                     constant_values=n)
        return jnp.stack([c0, c1])

    src5 = per_core(jnp.concatenate([src, pad]))
    dst5 = per_core(jnp.concatenate([dst, pad]))
    edges5 = jnp.concatenate([src5, dst5], axis=3)
    x_pad = jnp.pad(x, ((0, n_pad - n), (0, 0)))

    deg = _sc_deg(edges5, n_pad)
    hprime = _tc_h(x_pad, W, deg[0], deg[1], n_pad)
    parts = _sc_main(edges5, hprime, n_pad, nch0, nch1)
    out_pad = _tc_final(
        parts, hprime, deg[0], deg[1],
        b.reshape(1, D), prelu_weight.reshape(1, 1).astype(jnp.float32), n_pad,
    )
    return out_pad[:n]
